# baseline (speedup 1.0000x reference)
import functools

import jax
import jax.numpy as jnp
from jax import lax
from jax.experimental import pallas as pl
from jax.experimental.pallas import tpu as pltpu
from jax.experimental.pallas import tpu_sc as plsc

BATCH = 4096
HIST = 200
DIM = 32
INPUT_ROWS = 1000000
LANES = 16

NUM_CORES = 2
NUM_SUBCORES = 16
NW = NUM_CORES * NUM_SUBCORES
ROWS_PER_W = BATCH // NW

CHUNK0 = 128
CHUNK1 = HIST - CHUNK0

N_ACC = 4
SLAB = 4 * DIM


def _pooling_body(inputs_hbm, table2_hbm, out_hbm, idx_v, idxq_v, rows_v,
                  out_v, t0_v, sem0, sem1):
    cid = lax.axis_index("c")
    sid = lax.axis_index("s")
    wid = sid * NUM_CORES + cid
    base = wid * ROWS_PER_W

    pltpu.sync_copy(inputs_hbm.at[pl.ds(base, ROWS_PER_W), :], idx_v)
    pltpu.sync_copy(table2_hbm.at[pl.ds(0, 1), :], t0_v)
    t0a = t0_v[0, 0:LANES]
    t0b = t0_v[0, LANES:DIM]

    lane = lax.iota(jnp.int32, LANES)
    sems = (sem0, sem1)

    def compute_q(b, p):
        for k in range(HIST // LANES):
            c16 = idx_v[b, pl.ds(k * LANES, LANES)]
            idxq_v[p, pl.ds(k * LANES, LANES)] = c16 >> 2
        tail = idx_v[b, pl.ds(HIST - LANES, LANES)]
        idxq_v[p, pl.ds(HIST - LANES, LANES)] = tail >> 2

    def start_gather(b, p):
        compute_q(b, p)
        pltpu.make_async_copy(
            table2_hbm.at[idxq_v.at[p, pl.ds(0, CHUNK0)]],
            rows_v.at[p, pl.ds(0, CHUNK0), :], sems[p]).start()
        pltpu.make_async_copy(
            table2_hbm.at[idxq_v.at[p, pl.ds(CHUNK0, CHUNK1)]],
            rows_v.at[p, pl.ds(CHUNK0, CHUNK1), :], sems[p]).start()

    def wait_gather(p):
        pltpu.make_async_copy(
            table2_hbm.at[idxq_v.at[p, pl.ds(0, CHUNK0)]],
            rows_v.at[p, pl.ds(0, CHUNK0), :], sems[p]).wait()
        pltpu.make_async_copy(
            table2_hbm.at[idxq_v.at[p, pl.ds(CHUNK0, CHUNK1)]],
            rows_v.at[p, pl.ds(CHUNK0, CHUNK1), :], sems[p]).wait()

    def count_zeros(b):
        cz = jnp.zeros((LANES,), jnp.int32)
        for k in range(HIST // LANES):
            chunk = idx_v[b, pl.ds(k * LANES, LANES)]
            cz = cz + plsc.all_reduce_population_count(chunk == 0)
        tail = idx_v[b, pl.ds(HIST - LANES, LANES)]
        cz = cz + plsc.all_reduce_population_count(
            (tail == 0) & (lane >= LANES - (HIST % LANES)))
        return cz.astype(jnp.float32)

    def acc_chunk(p, b, k, accs, njj):
        accs = list(accs)
        c16 = idx_v[b, pl.ds(k * LANES, LANES)]
        o16 = (c16 & 3) << 5
        for jj in range(LANES - njj, LANES):
            j = k * LANES + jj
            o = pl.multiple_of(o16[jj], DIM)
            kk = jj % N_ACC
            accs[2 * kk] = accs[2 * kk] + rows_v[p, j, pl.ds(o, LANES)]
            accs[2 * kk + 1] = (accs[2 * kk + 1]
                                + rows_v[p, j, pl.ds(o + LANES, LANES)])
        return tuple(accs)

    def accumulate(p, b):
        zero = jnp.zeros((LANES,), jnp.float32)

        def step(k, ac):
            return acc_chunk(p, b, k, ac, LANES)

        accs = lax.fori_loop(0, HIST // LANES, step, (zero,) * 2 * N_ACC)
        accs = list(accs)
        c16 = idx_v[b, pl.ds(HIST - LANES, LANES)]
        o16 = (c16 & 3) << 5
        for jj in range(LANES - (HIST % LANES), LANES):
            j = HIST - LANES + jj
            o = pl.multiple_of(o16[jj], DIM)
            kk = jj % N_ACC
            accs[2 * kk] = accs[2 * kk] + rows_v[p, j, pl.ds(o, LANES)]
            accs[2 * kk + 1] = (accs[2 * kk + 1]
                                + rows_v[p, j, pl.ds(o + LANES, LANES)])
        a0 = accs[0]
        a1 = accs[1]
        for kk in range(1, N_ACC):
            a0 = a0 + accs[2 * kk]
            a1 = a1 + accs[2 * kk + 1]
        return a0, a1

    def finish_row(b, p):
        n0 = count_zeros(b)
        a0, a1 = accumulate(p, b)
        inv = 1.0 / jnp.maximum(jnp.float32(HIST) - n0, 1.0)
        out_v[b, 0:LANES] = (a0 - n0 * t0a) * inv
        out_v[b, LANES:DIM] = (a1 - n0 * t0b) * inv

    start_gather(0, 0)

    def pair_body(g, carry):
        b0 = g * 2
        start_gather(b0 + 1, 1)
        wait_gather(0)
        finish_row(b0, 0)

        @pl.when(g < ROWS_PER_W // 2 - 1)
        def _():
            start_gather(b0 + 2, 0)

        wait_gather(1)
        finish_row(b0 + 1, 1)
        return carry

    lax.fori_loop(0, ROWS_PER_W // 2, pair_body, 0)

    pltpu.sync_copy(out_v, out_hbm.at[pl.ds(base, ROWS_PER_W), :])


@functools.partial(
    pl.kernel,
    mesh=plsc.VectorSubcoreMesh(core_axis_name="c", subcore_axis_name="s"),
    compiler_params=pltpu.CompilerParams(needs_layout_passes=False,
                                         use_tc_tiling_on_sc=True),
    out_type=jax.ShapeDtypeStruct((BATCH, DIM), jnp.float32),
    scratch_types=[
        pltpu.VMEM((ROWS_PER_W, HIST), jnp.int32),
        pltpu.VMEM((2, HIST), jnp.int32),
        pltpu.VMEM((2, HIST, SLAB), jnp.float32),
        pltpu.VMEM((ROWS_PER_W, DIM), jnp.float32),
        pltpu.VMEM((1, SLAB), jnp.float32),
        pltpu.SemaphoreType.DMA,
        pltpu.SemaphoreType.DMA,
    ],
)
def _pooling_kernel(inputs_hbm, table2_hbm, out_hbm, idx_v, idxq_v, rows_v,
                    out_v, t0_v, sem0, sem1):
    _pooling_body(inputs_hbm, table2_hbm, out_hbm, idx_v, idxq_v, rows_v,
                  out_v, t0_v, sem0, sem1)


def kernel(inputs, table):
    return _pooling_kernel(inputs, table.reshape(INPUT_ROWS // 4, DIM * 4))


# speedup vs baseline: 1.3511x; 1.3511x over previous
import functools

import jax
import jax.numpy as jnp
from jax import lax
from jax.experimental import pallas as pl
from jax.experimental.pallas import tpu as pltpu
from jax.experimental.pallas import tpu_sc as plsc

BATCH = 4096
HIST = 200
DIM = 32
INPUT_ROWS = 1000000
LANES = 16

NUM_CORES = 2
NUM_SUBCORES = 16
NW = NUM_CORES * NUM_SUBCORES
ROWS_PER_W = BATCH // NW

CHUNK0 = 128
CHUNK1 = HIST - CHUNK0

N_ACC = 4

TCOLS = 2048
TGRID = -(-INPUT_ROWS // TCOLS)
TSLABS = TCOLS * DIM // 128


def _transpose_body(in_ref, out_ref):
    x = in_ref[...]
    pieces = [jnp.transpose(x[:, d2 * TSLABS:(d2 + 1) * TSLABS])
              for d2 in range(4)]
    out_ref[...] = jnp.concatenate(pieces, axis=1)


_transpose_tc = pl.pallas_call(
    _transpose_body,
    grid=(TGRID,),
    in_specs=[pl.BlockSpec((DIM, TCOLS), lambda i: (0, i))],
    out_specs=pl.BlockSpec((TSLABS, 128), lambda i: (i, 0)),
    out_shape=jax.ShapeDtypeStruct((TGRID * TSLABS, 128), jnp.float32),
)


def _pooling_body(inputs_hbm, table_hbm, out_hbm, idx_v, idxg_v, rows_v,
                  out_v, t0_v, sem0, sem1):
    cid = lax.axis_index("c")
    sid = lax.axis_index("s")
    wid = sid * NUM_CORES + cid
    base = wid * ROWS_PER_W

    pltpu.sync_copy(inputs_hbm.at[pl.ds(base, ROWS_PER_W), :], idx_v)
    pltpu.sync_copy(table_hbm.at[pl.ds(0, 1), :], t0_v)
    t0a = t0_v[0, 0:LANES]
    t0b = t0_v[0, LANES:DIM]

    lane = lax.iota(jnp.int32, LANES)
    sems = (sem0, sem1)

    def xform(t):
        return (((t >> 11) << 11) + ((t & 511) << 2) + ((t >> 9) & 3))

    def compute_g(b, p):
        for k in range(HIST // LANES):
            c16 = idx_v[b, pl.ds(k * LANES, LANES)]
            idxg_v[p, pl.ds(k * LANES, LANES)] = xform(c16)
        tail = idx_v[b, pl.ds(HIST - LANES, LANES)]
        idxg_v[p, pl.ds(HIST - LANES, LANES)] = xform(tail)

    def start_gather(b, p):
        compute_g(b, p)
        pltpu.make_async_copy(
            table_hbm.at[idxg_v.at[p, pl.ds(0, CHUNK0)]],
            rows_v.at[p, pl.ds(0, CHUNK0), :], sems[p]).start()
        pltpu.make_async_copy(
            table_hbm.at[idxg_v.at[p, pl.ds(CHUNK0, CHUNK1)]],
            rows_v.at[p, pl.ds(CHUNK0, CHUNK1), :], sems[p]).start()

    def wait_gather(p):
        pltpu.make_async_copy(
            table_hbm.at[idxg_v.at[p, pl.ds(0, CHUNK0)]],
            rows_v.at[p, pl.ds(0, CHUNK0), :], sems[p]).wait()
        pltpu.make_async_copy(
            table_hbm.at[idxg_v.at[p, pl.ds(CHUNK0, CHUNK1)]],
            rows_v.at[p, pl.ds(CHUNK0, CHUNK1), :], sems[p]).wait()

    def count_zeros(b):
        cz = jnp.zeros((LANES,), jnp.int32)
        for k in range(HIST // LANES):
            chunk = idx_v[b, pl.ds(k * LANES, LANES)]
            cz = cz + plsc.all_reduce_population_count(chunk == 0)
        tail = idx_v[b, pl.ds(HIST - LANES, LANES)]
        cz = cz + plsc.all_reduce_population_count(
            (tail == 0) & (lane >= LANES - (HIST % LANES)))
        return cz.astype(jnp.float32)

    def accumulate(p):
        zero = jnp.zeros((LANES,), jnp.float32)

        def step(s, ac):
            accs = list(ac)
            for jj in range(8):
                j = s * 8 + jj
                kk = jj % N_ACC
                accs[2 * kk] = accs[2 * kk] + rows_v[p, j, 0:LANES]
                accs[2 * kk + 1] = (accs[2 * kk + 1]
                                    + rows_v[p, j, LANES:DIM])
            return tuple(accs)

        accs = lax.fori_loop(0, HIST // 8, step, (zero,) * 2 * N_ACC)
        a0 = accs[0]
        a1 = accs[1]
        for kk in range(1, N_ACC):
            a0 = a0 + accs[2 * kk]
            a1 = a1 + accs[2 * kk + 1]
        return a0, a1

    def finish_row(b, p):
        n0 = count_zeros(b)
        a0, a1 = accumulate(p)
        inv = 1.0 / jnp.maximum(jnp.float32(HIST) - n0, 1.0)
        out_v[b, 0:LANES] = (a0 - n0 * t0a) * inv
        out_v[b, LANES:DIM] = (a1 - n0 * t0b) * inv

    start_gather(0, 0)

    def pair_body(g, carry):
        b0 = g * 2
        start_gather(b0 + 1, 1)
        wait_gather(0)
        finish_row(b0, 0)

        @pl.when(g < ROWS_PER_W // 2 - 1)
        def _():
            start_gather(b0 + 2, 0)

        wait_gather(1)
        finish_row(b0 + 1, 1)
        return carry

    lax.fori_loop(0, ROWS_PER_W // 2, pair_body, 0)

    pltpu.sync_copy(out_v, out_hbm.at[pl.ds(base, ROWS_PER_W), :])


@functools.partial(
    pl.kernel,
    mesh=plsc.VectorSubcoreMesh(core_axis_name="c", subcore_axis_name="s"),
    compiler_params=pltpu.CompilerParams(needs_layout_passes=False,
                                         use_tc_tiling_on_sc=False),
    out_type=jax.ShapeDtypeStruct((BATCH, DIM), jnp.float32),
    scratch_types=[
        pltpu.VMEM((ROWS_PER_W, HIST), jnp.int32),
        pltpu.VMEM((2, HIST), jnp.int32),
        pltpu.VMEM((2, HIST, DIM), jnp.float32),
        pltpu.VMEM((ROWS_PER_W, DIM), jnp.float32),
        pltpu.VMEM((1, DIM), jnp.float32),
        pltpu.SemaphoreType.DMA,
        pltpu.SemaphoreType.DMA,
    ],
)
def _pooling_kernel(inputs_hbm, table_hbm, out_hbm, idx_v, idxg_v, rows_v,
                    out_v, t0_v, sem0, sem1):
    _pooling_body(inputs_hbm, table_hbm, out_hbm, idx_v, idxg_v, rows_v,
                  out_v, t0_v, sem0, sem1)


def kernel(inputs, table):
    tab_slab = _transpose_tc(table.T)
    view = tab_slab.reshape(TGRID * TSLABS * 4, DIM)
    return _pooling_kernel(inputs, view)


# speedup vs baseline: 1.3522x; 1.0008x over previous
import functools

import jax
import jax.numpy as jnp
from jax import lax
from jax.experimental import pallas as pl
from jax.experimental.pallas import tpu as pltpu
from jax.experimental.pallas import tpu_sc as plsc

BATCH = 4096
HIST = 200
DIM = 32
INPUT_ROWS = 1000000
LANES = 16

NUM_CORES = 2
NUM_SUBCORES = 16
NW = NUM_CORES * NUM_SUBCORES
ROWS_PER_W = BATCH // NW

CHUNK0 = 128
CHUNK1 = HIST - CHUNK0

N_ACC = 4

TCOLS = 2048
TGRID = -(-INPUT_ROWS // TCOLS)
TSLABS = TCOLS * DIM // 128


def _transpose_body(in_ref, out_ref):
    x = in_ref[...]
    eye = jnp.eye(DIM, dtype=jnp.float32)
    y = lax.dot_general(x, eye, (((0,), (0,)), ((), ())),
                        preferred_element_type=jnp.float32)
    for d2 in range(4):
        out_ref[:, d2 * DIM:(d2 + 1) * DIM] = (
            y[d2 * TSLABS:(d2 + 1) * TSLABS, :])


_transpose_tc = pl.pallas_call(
    _transpose_body,
    grid=(TGRID,),
    in_specs=[pl.BlockSpec((DIM, TCOLS), lambda i: (0, i))],
    out_specs=pl.BlockSpec((TSLABS, 128), lambda i: (i, 0)),
    out_shape=jax.ShapeDtypeStruct((TGRID * TSLABS, 128), jnp.float32),
)


def _pooling_body(inputs_hbm, table_hbm, out_hbm, idx_v, idxg_v, rows_v,
                  out_v, t0_v, sem0, sem1):
    cid = lax.axis_index("c")
    sid = lax.axis_index("s")
    wid = sid * NUM_CORES + cid
    base = wid * ROWS_PER_W

    pltpu.sync_copy(inputs_hbm.at[pl.ds(base, ROWS_PER_W), :], idx_v)
    pltpu.sync_copy(table_hbm.at[pl.ds(0, 1), :], t0_v)
    t0a = t0_v[0, 0:LANES]
    t0b = t0_v[0, LANES:DIM]

    lane = lax.iota(jnp.int32, LANES)
    sems = (sem0, sem1)

    def xform(t):
        return (((t >> 11) << 11) + ((t & 511) << 2) + ((t >> 9) & 3))

    def compute_g(b, p):
        for k in range(HIST // LANES):
            c16 = idx_v[b, pl.ds(k * LANES, LANES)]
            idxg_v[p, pl.ds(k * LANES, LANES)] = xform(c16)
        tail = idx_v[b, pl.ds(HIST - LANES, LANES)]
        idxg_v[p, pl.ds(HIST - LANES, LANES)] = xform(tail)

    def start_gather(b, p):
        compute_g(b, p)
        pltpu.make_async_copy(
            table_hbm.at[idxg_v.at[p, pl.ds(0, CHUNK0)]],
            rows_v.at[p, pl.ds(0, CHUNK0), :], sems[p]).start()
        pltpu.make_async_copy(
            table_hbm.at[idxg_v.at[p, pl.ds(CHUNK0, CHUNK1)]],
            rows_v.at[p, pl.ds(CHUNK0, CHUNK1), :], sems[p]).start()

    def wait_gather(p):
        pltpu.make_async_copy(
            table_hbm.at[idxg_v.at[p, pl.ds(0, CHUNK0)]],
            rows_v.at[p, pl.ds(0, CHUNK0), :], sems[p]).wait()
        pltpu.make_async_copy(
            table_hbm.at[idxg_v.at[p, pl.ds(CHUNK0, CHUNK1)]],
            rows_v.at[p, pl.ds(CHUNK0, CHUNK1), :], sems[p]).wait()

    def count_zeros(b):
        cz = jnp.zeros((LANES,), jnp.int32)
        for k in range(HIST // LANES):
            chunk = idx_v[b, pl.ds(k * LANES, LANES)]
            cz = cz + plsc.all_reduce_population_count(chunk == 0)
        tail = idx_v[b, pl.ds(HIST - LANES, LANES)]
        cz = cz + plsc.all_reduce_population_count(
            (tail == 0) & (lane >= LANES - (HIST % LANES)))
        return cz.astype(jnp.float32)

    def accumulate(p):
        zero = jnp.zeros((LANES,), jnp.float32)

        def step(s, ac):
            accs = list(ac)
            for jj in range(8):
                j = s * 8 + jj
                kk = jj % N_ACC
                accs[2 * kk] = accs[2 * kk] + rows_v[p, j, 0:LANES]
                accs[2 * kk + 1] = (accs[2 * kk + 1]
                                    + rows_v[p, j, LANES:DIM])
            return tuple(accs)

        accs = lax.fori_loop(0, HIST // 8, step, (zero,) * 2 * N_ACC)
        a0 = accs[0]
        a1 = accs[1]
        for kk in range(1, N_ACC):
            a0 = a0 + accs[2 * kk]
            a1 = a1 + accs[2 * kk + 1]
        return a0, a1

    def finish_row(b, p):
        n0 = count_zeros(b)
        a0, a1 = accumulate(p)
        inv = 1.0 / jnp.maximum(jnp.float32(HIST) - n0, 1.0)
        out_v[b, 0:LANES] = (a0 - n0 * t0a) * inv
        out_v[b, LANES:DIM] = (a1 - n0 * t0b) * inv

    start_gather(0, 0)

    def pair_body(g, carry):
        b0 = g * 2
        start_gather(b0 + 1, 1)
        wait_gather(0)
        finish_row(b0, 0)

        @pl.when(g < ROWS_PER_W // 2 - 1)
        def _():
            start_gather(b0 + 2, 0)

        wait_gather(1)
        finish_row(b0 + 1, 1)
        return carry

    lax.fori_loop(0, ROWS_PER_W // 2, pair_body, 0)

    pltpu.sync_copy(out_v, out_hbm.at[pl.ds(base, ROWS_PER_W), :])


@functools.partial(
    pl.kernel,
    mesh=plsc.VectorSubcoreMesh(core_axis_name="c", subcore_axis_name="s"),
    compiler_params=pltpu.CompilerParams(needs_layout_passes=False,
                                         use_tc_tiling_on_sc=False),
    out_type=jax.ShapeDtypeStruct((BATCH, DIM), jnp.float32),
    scratch_types=[
        pltpu.VMEM((ROWS_PER_W, HIST), jnp.int32),
        pltpu.VMEM((2, HIST), jnp.int32),
        pltpu.VMEM((2, HIST, DIM), jnp.float32),
        pltpu.VMEM((ROWS_PER_W, DIM), jnp.float32),
        pltpu.VMEM((1, DIM), jnp.float32),
        pltpu.SemaphoreType.DMA,
        pltpu.SemaphoreType.DMA,
    ],
)
def _pooling_kernel(inputs_hbm, table_hbm, out_hbm, idx_v, idxg_v, rows_v,
                    out_v, t0_v, sem0, sem1):
    _pooling_body(inputs_hbm, table_hbm, out_hbm, idx_v, idxg_v, rows_v,
                  out_v, t0_v, sem0, sem1)


def kernel(inputs, table):
    tab_slab = _transpose_tc(table.T)
    view = tab_slab.reshape(TGRID * TSLABS * 4, DIM)
    return _pooling_kernel(inputs, view)


# speedup vs baseline: 1.9561x; 1.4467x over previous
import functools

import jax
import jax.numpy as jnp
from jax import lax
from jax.experimental import pallas as pl
from jax.experimental.pallas import tpu as pltpu
from jax.experimental.pallas import tpu_sc as plsc

BATCH = 4096
HIST = 200
DIM = 32
INPUT_ROWS = 1000000
LANES = 16

NUM_CORES = 2
NUM_SUBCORES = 16
NW = NUM_CORES * NUM_SUBCORES
ROWS_PER_W = BATCH // NW

CHUNK0 = 128
CHUNK1 = HIST - CHUNK0

N_ACC = 4

TCOLS = 8192
TGRID = -(-INPUT_ROWS // TCOLS)
TSLABS = TCOLS * DIM // 128
SH_B = TCOLS.bit_length() - 1
SH_S = TSLABS.bit_length() - 1


def _transpose_body(in_ref, out_ref):
    x = in_ref[...]
    eye = jnp.eye(DIM, dtype=jnp.float32)
    y = lax.dot_general(x, eye, (((0,), (0,)), ((), ())),
                        preferred_element_type=jnp.float32)
    for d2 in range(4):
        out_ref[:, d2 * DIM:(d2 + 1) * DIM] = (
            y[d2 * TSLABS:(d2 + 1) * TSLABS, :])


_transpose_tc = pl.pallas_call(
    _transpose_body,
    grid=(TGRID,),
    in_specs=[pl.BlockSpec((DIM, TCOLS), lambda i: (0, i))],
    out_specs=pl.BlockSpec((TSLABS, 128), lambda i: (i, 0)),
    out_shape=jax.ShapeDtypeStruct((TGRID * TSLABS, 128), jnp.float32),
)


def _pooling_body(inputs_hbm, table_hbm, out_hbm, idx_v, idxg_v, rows_v,
                  out_v, t0_v, sem0, sem1):
    cid = lax.axis_index("c")
    sid = lax.axis_index("s")
    wid = sid * NUM_CORES + cid
    base = wid * ROWS_PER_W

    pltpu.sync_copy(inputs_hbm.at[pl.ds(base, ROWS_PER_W), :], idx_v)
    pltpu.sync_copy(table_hbm.at[pl.ds(0, 1), :], t0_v)
    t0a = t0_v[0, 0:LANES]
    t0b = t0_v[0, LANES:DIM]

    lane = lax.iota(jnp.int32, LANES)
    sems = (sem0, sem1)

    def xform(t):
        return (((t >> SH_B) << SH_B) + ((t & (TSLABS - 1)) << 2)
                + ((t >> SH_S) & 3))

    def compute_g(b, p):
        for k in range(HIST // LANES):
            c16 = idx_v[b, pl.ds(k * LANES, LANES)]
            idxg_v[p, pl.ds(k * LANES, LANES)] = xform(c16)
        tail = idx_v[b, pl.ds(HIST - LANES, LANES)]
        idxg_v[p, pl.ds(HIST - LANES, LANES)] = xform(tail)

    def start_gather(b, p):
        compute_g(b, p)
        pltpu.make_async_copy(
            table_hbm.at[idxg_v.at[p, pl.ds(0, CHUNK0)]],
            rows_v.at[p, pl.ds(0, CHUNK0), :], sems[p]).start()
        pltpu.make_async_copy(
            table_hbm.at[idxg_v.at[p, pl.ds(CHUNK0, CHUNK1)]],
            rows_v.at[p, pl.ds(CHUNK0, CHUNK1), :], sems[p]).start()

    def wait_gather(p):
        pltpu.make_async_copy(
            table_hbm.at[idxg_v.at[p, pl.ds(0, CHUNK0)]],
            rows_v.at[p, pl.ds(0, CHUNK0), :], sems[p]).wait()
        pltpu.make_async_copy(
            table_hbm.at[idxg_v.at[p, pl.ds(CHUNK0, CHUNK1)]],
            rows_v.at[p, pl.ds(CHUNK0, CHUNK1), :], sems[p]).wait()

    def count_zeros(b):
        cz = jnp.zeros((LANES,), jnp.int32)
        for k in range(HIST // LANES):
            chunk = idx_v[b, pl.ds(k * LANES, LANES)]
            cz = cz + plsc.all_reduce_population_count(chunk == 0)
        tail = idx_v[b, pl.ds(HIST - LANES, LANES)]
        cz = cz + plsc.all_reduce_population_count(
            (tail == 0) & (lane >= LANES - (HIST % LANES)))
        return cz.astype(jnp.float32)

    def accumulate(p):
        zero = jnp.zeros((LANES,), jnp.float32)

        def step(s, ac):
            accs = list(ac)
            for jj in range(8):
                j = s * 8 + jj
                kk = jj % N_ACC
                accs[2 * kk] = accs[2 * kk] + rows_v[p, j, 0:LANES]
                accs[2 * kk + 1] = (accs[2 * kk + 1]
                                    + rows_v[p, j, LANES:DIM])
            return tuple(accs)

        accs = lax.fori_loop(0, HIST // 8, step, (zero,) * 2 * N_ACC)
        a0 = accs[0]
        a1 = accs[1]
        for kk in range(1, N_ACC):
            a0 = a0 + accs[2 * kk]
            a1 = a1 + accs[2 * kk + 1]
        return a0, a1

    def finish_row(b, p):
        n0 = count_zeros(b)
        a0, a1 = accumulate(p)
        inv = 1.0 / jnp.maximum(jnp.float32(HIST) - n0, 1.0)
        out_v[b, 0:LANES] = (a0 - n0 * t0a) * inv
        out_v[b, LANES:DIM] = (a1 - n0 * t0b) * inv

    start_gather(0, 0)

    def pair_body(g, carry):
        b0 = g * 2
        start_gather(b0 + 1, 1)
        wait_gather(0)
        finish_row(b0, 0)

        @pl.when(g < ROWS_PER_W // 2 - 1)
        def _():
            start_gather(b0 + 2, 0)

        wait_gather(1)
        finish_row(b0 + 1, 1)
        return carry

    lax.fori_loop(0, ROWS_PER_W // 2, pair_body, 0)

    pltpu.sync_copy(out_v, out_hbm.at[pl.ds(base, ROWS_PER_W), :])


@functools.partial(
    pl.kernel,
    mesh=plsc.VectorSubcoreMesh(core_axis_name="c", subcore_axis_name="s"),
    compiler_params=pltpu.CompilerParams(needs_layout_passes=False,
                                         use_tc_tiling_on_sc=False),
    out_type=jax.ShapeDtypeStruct((BATCH, DIM), jnp.float32),
    scratch_types=[
        pltpu.VMEM((ROWS_PER_W, HIST), jnp.int32),
        pltpu.VMEM((2, HIST), jnp.int32),
        pltpu.VMEM((2, HIST, DIM), jnp.float32),
        pltpu.VMEM((ROWS_PER_W, DIM), jnp.float32),
        pltpu.VMEM((1, DIM), jnp.float32),
        pltpu.SemaphoreType.DMA,
        pltpu.SemaphoreType.DMA,
    ],
)
def _pooling_kernel(inputs_hbm, table_hbm, out_hbm, idx_v, idxg_v, rows_v,
                    out_v, t0_v, sem0, sem1):
    _pooling_body(inputs_hbm, table_hbm, out_hbm, idx_v, idxg_v, rows_v,
                  out_v, t0_v, sem0, sem1)


def kernel(inputs, table):
    tab_slab = _transpose_tc(table.T)
    view = tab_slab.reshape(TGRID * TSLABS * 4, DIM)
    return _pooling_kernel(inputs, view)


# speedup vs baseline: 1.9883x; 1.0164x over previous
import functools

import jax
import jax.numpy as jnp
from jax import lax
from jax.experimental import pallas as pl
from jax.experimental.pallas import tpu as pltpu
from jax.experimental.pallas import tpu_sc as plsc

BATCH = 4096
HIST = 200
DIM = 32
INPUT_ROWS = 1000000
LANES = 16

NUM_CORES = 2
NUM_SUBCORES = 16
NW = NUM_CORES * NUM_SUBCORES
ROWS_PER_W = BATCH // NW

CHUNK0 = 128
CHUNK1 = HIST - CHUNK0

N_ACC = 4

TCOLS = 32768
TGRID = -(-INPUT_ROWS // TCOLS)
TSLABS = TCOLS * DIM // 128
SH_B = TCOLS.bit_length() - 1
SH_S = TSLABS.bit_length() - 1


def _transpose_body(in_ref, out_ref):
    x = in_ref[...]
    eye = jnp.eye(DIM, dtype=jnp.float32)
    y = lax.dot_general(x, eye, (((0,), (0,)), ((), ())),
                        preferred_element_type=jnp.float32)
    for d2 in range(4):
        out_ref[:, d2 * DIM:(d2 + 1) * DIM] = (
            y[d2 * TSLABS:(d2 + 1) * TSLABS, :])


_transpose_tc = pl.pallas_call(
    _transpose_body,
    grid=(TGRID,),
    in_specs=[pl.BlockSpec((DIM, TCOLS), lambda i: (0, i))],
    out_specs=pl.BlockSpec((TSLABS, 128), lambda i: (i, 0)),
    out_shape=jax.ShapeDtypeStruct((TGRID * TSLABS, 128), jnp.float32),
)


def _pooling_body(inputs_hbm, table_hbm, out_hbm, idx_v, idxg_v, rows_v,
                  out_v, t0_v, sem0, sem1):
    cid = lax.axis_index("c")
    sid = lax.axis_index("s")
    wid = sid * NUM_CORES + cid
    base = wid * ROWS_PER_W

    pltpu.sync_copy(inputs_hbm.at[pl.ds(base, ROWS_PER_W), :], idx_v)
    pltpu.sync_copy(table_hbm.at[pl.ds(0, 1), :], t0_v)
    t0a = t0_v[0, 0:LANES]
    t0b = t0_v[0, LANES:DIM]

    lane = lax.iota(jnp.int32, LANES)
    sems = (sem0, sem1)

    def xform(t):
        return (((t >> SH_B) << SH_B) + ((t & (TSLABS - 1)) << 2)
                + ((t >> SH_S) & 3))

    def compute_g(b, p):
        for k in range(HIST // LANES):
            c16 = idx_v[b, pl.ds(k * LANES, LANES)]
            idxg_v[p, pl.ds(k * LANES, LANES)] = xform(c16)
        tail = idx_v[b, pl.ds(HIST - LANES, LANES)]
        idxg_v[p, pl.ds(HIST - LANES, LANES)] = xform(tail)

    def start_gather(b, p):
        compute_g(b, p)
        pltpu.make_async_copy(
            table_hbm.at[idxg_v.at[p, pl.ds(0, CHUNK0)]],
            rows_v.at[p, pl.ds(0, CHUNK0), :], sems[p]).start()
        pltpu.make_async_copy(
            table_hbm.at[idxg_v.at[p, pl.ds(CHUNK0, CHUNK1)]],
            rows_v.at[p, pl.ds(CHUNK0, CHUNK1), :], sems[p]).start()

    def wait_gather(p):
        pltpu.make_async_copy(
            table_hbm.at[idxg_v.at[p, pl.ds(0, CHUNK0)]],
            rows_v.at[p, pl.ds(0, CHUNK0), :], sems[p]).wait()
        pltpu.make_async_copy(
            table_hbm.at[idxg_v.at[p, pl.ds(CHUNK0, CHUNK1)]],
            rows_v.at[p, pl.ds(CHUNK0, CHUNK1), :], sems[p]).wait()

    def count_zeros(b):
        cz = jnp.zeros((LANES,), jnp.int32)
        for k in range(HIST // LANES):
            chunk = idx_v[b, pl.ds(k * LANES, LANES)]
            cz = cz + plsc.all_reduce_population_count(chunk == 0)
        tail = idx_v[b, pl.ds(HIST - LANES, LANES)]
        cz = cz + plsc.all_reduce_population_count(
            (tail == 0) & (lane >= LANES - (HIST % LANES)))
        return cz.astype(jnp.float32)

    def accumulate(p):
        zero = jnp.zeros((LANES,), jnp.float32)

        def step(s, ac):
            accs = list(ac)
            for jj in range(8):
                j = s * 8 + jj
                kk = jj % N_ACC
                accs[2 * kk] = accs[2 * kk] + rows_v[p, j, 0:LANES]
                accs[2 * kk + 1] = (accs[2 * kk + 1]
                                    + rows_v[p, j, LANES:DIM])
            return tuple(accs)

        accs = lax.fori_loop(0, HIST // 8, step, (zero,) * 2 * N_ACC)
        a0 = accs[0]
        a1 = accs[1]
        for kk in range(1, N_ACC):
            a0 = a0 + accs[2 * kk]
            a1 = a1 + accs[2 * kk + 1]
        return a0, a1

    def finish_row(b, p):
        n0 = count_zeros(b)
        a0, a1 = accumulate(p)
        inv = 1.0 / jnp.maximum(jnp.float32(HIST) - n0, 1.0)
        out_v[b, 0:LANES] = (a0 - n0 * t0a) * inv
        out_v[b, LANES:DIM] = (a1 - n0 * t0b) * inv

    start_gather(0, 0)

    def pair_body(g, carry):
        b0 = g * 2
        start_gather(b0 + 1, 1)
        wait_gather(0)
        finish_row(b0, 0)

        @pl.when(g < ROWS_PER_W // 2 - 1)
        def _():
            start_gather(b0 + 2, 0)

        wait_gather(1)
        finish_row(b0 + 1, 1)
        return carry

    lax.fori_loop(0, ROWS_PER_W // 2, pair_body, 0)

    pltpu.sync_copy(out_v, out_hbm.at[pl.ds(base, ROWS_PER_W), :])


@functools.partial(
    pl.kernel,
    mesh=plsc.VectorSubcoreMesh(core_axis_name="c", subcore_axis_name="s"),
    compiler_params=pltpu.CompilerParams(needs_layout_passes=False,
                                         use_tc_tiling_on_sc=False),
    out_type=jax.ShapeDtypeStruct((BATCH, DIM), jnp.float32),
    scratch_types=[
        pltpu.VMEM((ROWS_PER_W, HIST), jnp.int32),
        pltpu.VMEM((2, HIST), jnp.int32),
        pltpu.VMEM((2, HIST, DIM), jnp.float32),
        pltpu.VMEM((ROWS_PER_W, DIM), jnp.float32),
        pltpu.VMEM((1, DIM), jnp.float32),
        pltpu.SemaphoreType.DMA,
        pltpu.SemaphoreType.DMA,
    ],
)
def _pooling_kernel(inputs_hbm, table_hbm, out_hbm, idx_v, idxg_v, rows_v,
                    out_v, t0_v, sem0, sem1):
    _pooling_body(inputs_hbm, table_hbm, out_hbm, idx_v, idxg_v, rows_v,
                  out_v, t0_v, sem0, sem1)


def kernel(inputs, table):
    tab_slab = _transpose_tc(table.T)
    view = tab_slab.reshape(TGRID * TSLABS * 4, DIM)
    return _pooling_kernel(inputs, view)


# speedup vs baseline: 2.7306x; 1.3733x over previous
import functools

import jax
import jax.numpy as jnp
from jax import lax
from jax.experimental import pallas as pl
from jax.experimental.pallas import tpu as pltpu
from jax.experimental.pallas import tpu_sc as plsc

BATCH = 4096
HIST = 200
DIM = 32
INPUT_ROWS = 1000000
LANES = 16

NUM_CORES = 2
NUM_SUBCORES = 16
NW = NUM_CORES * NUM_SUBCORES
ROWS_PER_W = BATCH // NW

CHUNK0 = 128
CHUNK1 = HIST - CHUNK0

N_ACC = 4

TCOLS = 32768
TGRID = -(-INPUT_ROWS // TCOLS)
TSLABS = TCOLS * DIM // 128
SH_B = TCOLS.bit_length() - 1
SH_S = TSLABS.bit_length() - 1


def _transpose_body(in_ref, out_ref):
    row = lax.broadcasted_iota(jnp.int32, (DIM, 128), 0)
    col = lax.broadcasted_iota(jnp.int32, (DIM, 128), 1)
    acc = None
    for d2 in range(4):
        x_d2 = in_ref[:, d2 * TSLABS:(d2 + 1) * TSLABS]
        e_d2 = (col == d2 * DIM + row).astype(jnp.float32)
        z = lax.dot_general(x_d2, e_d2, (((0,), (0,)), ((), ())),
                            preferred_element_type=jnp.float32)
        acc = z if acc is None else acc + z
    out_ref[...] = acc


_transpose_tc = pl.pallas_call(
    _transpose_body,
    grid=(TGRID,),
    in_specs=[pl.BlockSpec((DIM, TCOLS), lambda i: (0, i))],
    out_specs=pl.BlockSpec((TSLABS, 128), lambda i: (i, 0)),
    out_shape=jax.ShapeDtypeStruct((TGRID * TSLABS, 128), jnp.float32),
)


def _pooling_body(inputs_hbm, table_hbm, out_hbm, idx_v, idxg_v, rows_v,
                  out_v, t0_v, sem0, sem1):
    cid = lax.axis_index("c")
    sid = lax.axis_index("s")
    wid = sid * NUM_CORES + cid
    base = wid * ROWS_PER_W

    pltpu.sync_copy(inputs_hbm.at[pl.ds(base, ROWS_PER_W), :], idx_v)
    pltpu.sync_copy(table_hbm.at[pl.ds(0, 1), :], t0_v)
    t0a = t0_v[0, 0:LANES]
    t0b = t0_v[0, LANES:DIM]

    lane = lax.iota(jnp.int32, LANES)
    sems = (sem0, sem1)

    def xform(t):
        return (((t >> SH_B) << SH_B) + ((t & (TSLABS - 1)) << 2)
                + ((t >> SH_S) & 3))

    def compute_g(b, p):
        for k in range(HIST // LANES):
            c16 = idx_v[b, pl.ds(k * LANES, LANES)]
            idxg_v[p, pl.ds(k * LANES, LANES)] = xform(c16)
        tail = idx_v[b, pl.ds(HIST - LANES, LANES)]
        idxg_v[p, pl.ds(HIST - LANES, LANES)] = xform(tail)

    def start_gather(b, p):
        compute_g(b, p)
        pltpu.make_async_copy(
            table_hbm.at[idxg_v.at[p, pl.ds(0, CHUNK0)]],
            rows_v.at[p, pl.ds(0, CHUNK0), :], sems[p]).start()
        pltpu.make_async_copy(
            table_hbm.at[idxg_v.at[p, pl.ds(CHUNK0, CHUNK1)]],
            rows_v.at[p, pl.ds(CHUNK0, CHUNK1), :], sems[p]).start()

    def wait_gather(p):
        pltpu.make_async_copy(
            table_hbm.at[idxg_v.at[p, pl.ds(0, CHUNK0)]],
            rows_v.at[p, pl.ds(0, CHUNK0), :], sems[p]).wait()
        pltpu.make_async_copy(
            table_hbm.at[idxg_v.at[p, pl.ds(CHUNK0, CHUNK1)]],
            rows_v.at[p, pl.ds(CHUNK0, CHUNK1), :], sems[p]).wait()

    def count_zeros(b):
        cz = jnp.zeros((LANES,), jnp.int32)
        for k in range(HIST // LANES):
            chunk = idx_v[b, pl.ds(k * LANES, LANES)]
            cz = cz + plsc.all_reduce_population_count(chunk == 0)
        tail = idx_v[b, pl.ds(HIST - LANES, LANES)]
        cz = cz + plsc.all_reduce_population_count(
            (tail == 0) & (lane >= LANES - (HIST % LANES)))
        return cz.astype(jnp.float32)

    def accumulate(p):
        zero = jnp.zeros((LANES,), jnp.float32)

        def step(s, ac):
            accs = list(ac)
            for jj in range(8):
                j = s * 8 + jj
                kk = jj % N_ACC
                accs[2 * kk] = accs[2 * kk] + rows_v[p, j, 0:LANES]
                accs[2 * kk + 1] = (accs[2 * kk + 1]
                                    + rows_v[p, j, LANES:DIM])
            return tuple(accs)

        accs = lax.fori_loop(0, HIST // 8, step, (zero,) * 2 * N_ACC)
        a0 = accs[0]
        a1 = accs[1]
        for kk in range(1, N_ACC):
            a0 = a0 + accs[2 * kk]
            a1 = a1 + accs[2 * kk + 1]
        return a0, a1

    def finish_row(b, p):
        n0 = count_zeros(b)
        a0, a1 = accumulate(p)
        inv = 1.0 / jnp.maximum(jnp.float32(HIST) - n0, 1.0)
        out_v[b, 0:LANES] = (a0 - n0 * t0a) * inv
        out_v[b, LANES:DIM] = (a1 - n0 * t0b) * inv

    start_gather(0, 0)

    def pair_body(g, carry):
        b0 = g * 2
        start_gather(b0 + 1, 1)
        wait_gather(0)
        finish_row(b0, 0)

        @pl.when(g < ROWS_PER_W // 2 - 1)
        def _():
            start_gather(b0 + 2, 0)

        wait_gather(1)
        finish_row(b0 + 1, 1)
        return carry

    lax.fori_loop(0, ROWS_PER_W // 2, pair_body, 0)

    pltpu.sync_copy(out_v, out_hbm.at[pl.ds(base, ROWS_PER_W), :])


@functools.partial(
    pl.kernel,
    mesh=plsc.VectorSubcoreMesh(core_axis_name="c", subcore_axis_name="s"),
    compiler_params=pltpu.CompilerParams(needs_layout_passes=False,
                                         use_tc_tiling_on_sc=False),
    out_type=jax.ShapeDtypeStruct((BATCH, DIM), jnp.float32),
    scratch_types=[
        pltpu.VMEM((ROWS_PER_W, HIST), jnp.int32),
        pltpu.VMEM((2, HIST), jnp.int32),
        pltpu.VMEM((2, HIST, DIM), jnp.float32),
        pltpu.VMEM((ROWS_PER_W, DIM), jnp.float32),
        pltpu.VMEM((1, DIM), jnp.float32),
        pltpu.SemaphoreType.DMA,
        pltpu.SemaphoreType.DMA,
    ],
)
def _pooling_kernel(inputs_hbm, table_hbm, out_hbm, idx_v, idxg_v, rows_v,
                    out_v, t0_v, sem0, sem1):
    _pooling_body(inputs_hbm, table_hbm, out_hbm, idx_v, idxg_v, rows_v,
                  out_v, t0_v, sem0, sem1)


def kernel(inputs, table):
    tab_slab = _transpose_tc(table.T)
    view = tab_slab.reshape(TGRID * TSLABS * 4, DIM)
    return _pooling_kernel(inputs, view)


# speedup vs baseline: 3.3045x; 1.2102x over previous
import functools

import jax
import jax.numpy as jnp
from jax import lax
from jax.experimental import pallas as pl
from jax.experimental.pallas import tpu as pltpu
from jax.experimental.pallas import tpu_sc as plsc

BATCH = 4096
HIST = 200
DIM = 32
INPUT_ROWS = 1000000
LANES = 16

NUM_CORES = 2
NUM_SUBCORES = 16
NW = NUM_CORES * NUM_SUBCORES
ROWS_PER_W = BATCH // NW

CHUNK0 = 128
CHUNK1 = HIST - CHUNK0

N_ACC = 4

TCOLS = 32768
TGRID = -(-INPUT_ROWS // TCOLS)
TSLABS = TCOLS * DIM // 128
SH_B = TCOLS.bit_length() - 1
SH_S = TSLABS.bit_length() - 1


def _transpose_body(in_ref, out_ref):
    row = lax.broadcasted_iota(jnp.int32, (DIM, 128), 0)
    col = lax.broadcasted_iota(jnp.int32, (DIM, 128), 1)
    acc = None
    for d2 in range(4):
        x_d2 = in_ref[:, d2 * TSLABS:(d2 + 1) * TSLABS]
        e_d2 = (col == d2 * DIM + row).astype(jnp.bfloat16)
        z = lax.dot_general(x_d2.astype(jnp.bfloat16), e_d2,
                            (((0,), (0,)), ((), ())),
                            preferred_element_type=jnp.float32)
        acc = z if acc is None else acc + z
    out_ref[...] = acc


_transpose_tc = pl.pallas_call(
    _transpose_body,
    grid=(TGRID,),
    in_specs=[pl.BlockSpec((DIM, TCOLS), lambda i: (0, i))],
    out_specs=pl.BlockSpec((TSLABS, 128), lambda i: (i, 0)),
    out_shape=jax.ShapeDtypeStruct((TGRID * TSLABS, 128), jnp.float32),
)


def _pooling_body(inputs_hbm, table_hbm, out_hbm, idx_v, idxg_v, rows_v,
                  out_v, t0_v, sem0, sem1):
    cid = lax.axis_index("c")
    sid = lax.axis_index("s")
    wid = sid * NUM_CORES + cid
    base = wid * ROWS_PER_W

    pltpu.sync_copy(inputs_hbm.at[pl.ds(base, ROWS_PER_W), :], idx_v)
    pltpu.sync_copy(table_hbm.at[pl.ds(0, 1), :], t0_v)
    t0a = t0_v[0, 0:LANES]
    t0b = t0_v[0, LANES:DIM]

    lane = lax.iota(jnp.int32, LANES)
    sems = (sem0, sem1)

    def xform(t):
        return (((t >> SH_B) << SH_B) + ((t & (TSLABS - 1)) << 2)
                + ((t >> SH_S) & 3))

    def compute_g(b, p):
        for k in range(HIST // LANES):
            c16 = idx_v[b, pl.ds(k * LANES, LANES)]
            idxg_v[p, pl.ds(k * LANES, LANES)] = xform(c16)
        tail = idx_v[b, pl.ds(HIST - LANES, LANES)]
        idxg_v[p, pl.ds(HIST - LANES, LANES)] = xform(tail)

    def start_gather(b, p):
        compute_g(b, p)
        pltpu.make_async_copy(
            table_hbm.at[idxg_v.at[p, pl.ds(0, CHUNK0)]],
            rows_v.at[p, pl.ds(0, CHUNK0), :], sems[p]).start()
        pltpu.make_async_copy(
            table_hbm.at[idxg_v.at[p, pl.ds(CHUNK0, CHUNK1)]],
            rows_v.at[p, pl.ds(CHUNK0, CHUNK1), :], sems[p]).start()

    def wait_gather(p):
        pltpu.make_async_copy(
            table_hbm.at[idxg_v.at[p, pl.ds(0, CHUNK0)]],
            rows_v.at[p, pl.ds(0, CHUNK0), :], sems[p]).wait()
        pltpu.make_async_copy(
            table_hbm.at[idxg_v.at[p, pl.ds(CHUNK0, CHUNK1)]],
            rows_v.at[p, pl.ds(CHUNK0, CHUNK1), :], sems[p]).wait()

    def count_zeros(b):
        cz = jnp.zeros((LANES,), jnp.int32)
        for k in range(HIST // LANES):
            chunk = idx_v[b, pl.ds(k * LANES, LANES)]
            cz = cz + plsc.all_reduce_population_count(chunk == 0)
        tail = idx_v[b, pl.ds(HIST - LANES, LANES)]
        cz = cz + plsc.all_reduce_population_count(
            (tail == 0) & (lane >= LANES - (HIST % LANES)))
        return cz.astype(jnp.float32)

    def accumulate(p):
        zero = jnp.zeros((LANES,), jnp.float32)

        def step(s, ac):
            accs = list(ac)
            for jj in range(8):
                j = s * 8 + jj
                kk = jj % N_ACC
                accs[2 * kk] = accs[2 * kk] + rows_v[p, j, 0:LANES]
                accs[2 * kk + 1] = (accs[2 * kk + 1]
                                    + rows_v[p, j, LANES:DIM])
            return tuple(accs)

        accs = lax.fori_loop(0, HIST // 8, step, (zero,) * 2 * N_ACC)
        a0 = accs[0]
        a1 = accs[1]
        for kk in range(1, N_ACC):
            a0 = a0 + accs[2 * kk]
            a1 = a1 + accs[2 * kk + 1]
        return a0, a1

    def finish_row(b, p):
        n0 = count_zeros(b)
        a0, a1 = accumulate(p)
        inv = 1.0 / jnp.maximum(jnp.float32(HIST) - n0, 1.0)
        out_v[b, 0:LANES] = (a0 - n0 * t0a) * inv
        out_v[b, LANES:DIM] = (a1 - n0 * t0b) * inv

    start_gather(0, 0)

    def pair_body(g, carry):
        b0 = g * 2
        start_gather(b0 + 1, 1)
        wait_gather(0)
        finish_row(b0, 0)

        @pl.when(g < ROWS_PER_W // 2 - 1)
        def _():
            start_gather(b0 + 2, 0)

        wait_gather(1)
        finish_row(b0 + 1, 1)
        return carry

    lax.fori_loop(0, ROWS_PER_W // 2, pair_body, 0)

    pltpu.sync_copy(out_v, out_hbm.at[pl.ds(base, ROWS_PER_W), :])


@functools.partial(
    pl.kernel,
    mesh=plsc.VectorSubcoreMesh(core_axis_name="c", subcore_axis_name="s"),
    compiler_params=pltpu.CompilerParams(needs_layout_passes=False,
                                         use_tc_tiling_on_sc=False),
    out_type=jax.ShapeDtypeStruct((BATCH, DIM), jnp.float32),
    scratch_types=[
        pltpu.VMEM((ROWS_PER_W, HIST), jnp.int32),
        pltpu.VMEM((2, HIST), jnp.int32),
        pltpu.VMEM((2, HIST, DIM), jnp.float32),
        pltpu.VMEM((ROWS_PER_W, DIM), jnp.float32),
        pltpu.VMEM((1, DIM), jnp.float32),
        pltpu.SemaphoreType.DMA,
        pltpu.SemaphoreType.DMA,
    ],
)
def _pooling_kernel(inputs_hbm, table_hbm, out_hbm, idx_v, idxg_v, rows_v,
                    out_v, t0_v, sem0, sem1):
    _pooling_body(inputs_hbm, table_hbm, out_hbm, idx_v, idxg_v, rows_v,
                  out_v, t0_v, sem0, sem1)


def kernel(inputs, table):
    tab_slab = _transpose_tc(table.T)
    view = tab_slab.reshape(TGRID * TSLABS * 4, DIM)
    return _pooling_kernel(inputs, view)


# speedup vs baseline: 3.3938x; 1.0270x over previous
import functools

import jax
import jax.numpy as jnp
from jax import lax
from jax.experimental import pallas as pl
from jax.experimental.pallas import tpu as pltpu
from jax.experimental.pallas import tpu_sc as plsc

BATCH = 4096
HIST = 200
DIM = 32
INPUT_ROWS = 1000000
LANES = 16

NUM_CORES = 2
NUM_SUBCORES = 16
NW = NUM_CORES * NUM_SUBCORES
ROWS_PER_W = BATCH // NW

CHUNK0 = 128
CHUNK1 = HIST - CHUNK0

N_ACC = 4

TCOLS = 65536
TGRID = -(-INPUT_ROWS // TCOLS)
TSLABS = TCOLS * DIM // 128
SH_B = TCOLS.bit_length() - 1
SH_S = TSLABS.bit_length() - 1


def _transpose_body(in_ref, out_ref):
    row = lax.broadcasted_iota(jnp.int32, (DIM, 128), 0)
    col = lax.broadcasted_iota(jnp.int32, (DIM, 128), 1)
    acc = None
    for d2 in range(4):
        x_d2 = in_ref[:, d2 * TSLABS:(d2 + 1) * TSLABS]
        e_d2 = (col == d2 * DIM + row).astype(jnp.bfloat16)
        z = lax.dot_general(x_d2.astype(jnp.bfloat16), e_d2,
                            (((0,), (0,)), ((), ())),
                            preferred_element_type=jnp.float32)
        acc = z if acc is None else acc + z
    out_ref[...] = acc


_transpose_tc = pl.pallas_call(
    _transpose_body,
    grid=(TGRID,),
    in_specs=[pl.BlockSpec((DIM, TCOLS), lambda i: (0, i))],
    out_specs=pl.BlockSpec((TSLABS, 128), lambda i: (i, 0)),
    out_shape=jax.ShapeDtypeStruct((TGRID * TSLABS, 128), jnp.float32),
)


def _pooling_body(inputs_hbm, table_hbm, out_hbm, idx_v, idxg_v, rows_v,
                  out_v, t0_v, sem0, sem1):
    cid = lax.axis_index("c")
    sid = lax.axis_index("s")
    wid = sid * NUM_CORES + cid
    base = wid * ROWS_PER_W

    pltpu.sync_copy(inputs_hbm.at[pl.ds(base, ROWS_PER_W), :], idx_v)
    pltpu.sync_copy(table_hbm.at[pl.ds(0, 1), :], t0_v)
    t0a = t0_v[0, 0:LANES]
    t0b = t0_v[0, LANES:DIM]

    lane = lax.iota(jnp.int32, LANES)
    sems = (sem0, sem1)

    def xform(t):
        return (((t >> SH_B) << SH_B) + ((t & (TSLABS - 1)) << 2)
                + ((t >> SH_S) & 3))

    def compute_g(b, p):
        for k in range(HIST // LANES):
            c16 = idx_v[b, pl.ds(k * LANES, LANES)]
            idxg_v[p, pl.ds(k * LANES, LANES)] = xform(c16)
        tail = idx_v[b, pl.ds(HIST - LANES, LANES)]
        idxg_v[p, pl.ds(HIST - LANES, LANES)] = xform(tail)

    def start_gather(b, p):
        compute_g(b, p)
        pltpu.make_async_copy(
            table_hbm.at[idxg_v.at[p, pl.ds(0, CHUNK0)]],
            rows_v.at[p, pl.ds(0, CHUNK0), :], sems[p]).start()
        pltpu.make_async_copy(
            table_hbm.at[idxg_v.at[p, pl.ds(CHUNK0, CHUNK1)]],
            rows_v.at[p, pl.ds(CHUNK0, CHUNK1), :], sems[p]).start()

    def wait_gather(p):
        pltpu.make_async_copy(
            table_hbm.at[idxg_v.at[p, pl.ds(0, CHUNK0)]],
            rows_v.at[p, pl.ds(0, CHUNK0), :], sems[p]).wait()
        pltpu.make_async_copy(
            table_hbm.at[idxg_v.at[p, pl.ds(CHUNK0, CHUNK1)]],
            rows_v.at[p, pl.ds(CHUNK0, CHUNK1), :], sems[p]).wait()

    def count_zeros(b):
        cz = jnp.zeros((LANES,), jnp.int32)
        for k in range(HIST // LANES):
            chunk = idx_v[b, pl.ds(k * LANES, LANES)]
            cz = cz + plsc.all_reduce_population_count(chunk == 0)
        tail = idx_v[b, pl.ds(HIST - LANES, LANES)]
        cz = cz + plsc.all_reduce_population_count(
            (tail == 0) & (lane >= LANES - (HIST % LANES)))
        return cz.astype(jnp.float32)

    def accumulate(p):
        zero = jnp.zeros((LANES,), jnp.float32)

        def step(s, ac):
            accs = list(ac)
            for jj in range(8):
                j = s * 8 + jj
                kk = jj % N_ACC
                accs[2 * kk] = accs[2 * kk] + rows_v[p, j, 0:LANES]
                accs[2 * kk + 1] = (accs[2 * kk + 1]
                                    + rows_v[p, j, LANES:DIM])
            return tuple(accs)

        accs = lax.fori_loop(0, HIST // 8, step, (zero,) * 2 * N_ACC)
        a0 = accs[0]
        a1 = accs[1]
        for kk in range(1, N_ACC):
            a0 = a0 + accs[2 * kk]
            a1 = a1 + accs[2 * kk + 1]
        return a0, a1

    def finish_row(b, p):
        n0 = count_zeros(b)
        a0, a1 = accumulate(p)
        inv = 1.0 / jnp.maximum(jnp.float32(HIST) - n0, 1.0)
        out_v[b, 0:LANES] = (a0 - n0 * t0a) * inv
        out_v[b, LANES:DIM] = (a1 - n0 * t0b) * inv

    start_gather(0, 0)

    def pair_body(g, carry):
        b0 = g * 2
        start_gather(b0 + 1, 1)
        wait_gather(0)
        finish_row(b0, 0)

        @pl.when(g < ROWS_PER_W // 2 - 1)
        def _():
            start_gather(b0 + 2, 0)

        wait_gather(1)
        finish_row(b0 + 1, 1)
        return carry

    lax.fori_loop(0, ROWS_PER_W // 2, pair_body, 0)

    pltpu.sync_copy(out_v, out_hbm.at[pl.ds(base, ROWS_PER_W), :])


@functools.partial(
    pl.kernel,
    mesh=plsc.VectorSubcoreMesh(core_axis_name="c", subcore_axis_name="s"),
    compiler_params=pltpu.CompilerParams(needs_layout_passes=False,
                                         use_tc_tiling_on_sc=False),
    out_type=jax.ShapeDtypeStruct((BATCH, DIM), jnp.float32),
    scratch_types=[
        pltpu.VMEM((ROWS_PER_W, HIST), jnp.int32),
        pltpu.VMEM((2, HIST), jnp.int32),
        pltpu.VMEM((2, HIST, DIM), jnp.float32),
        pltpu.VMEM((ROWS_PER_W, DIM), jnp.float32),
        pltpu.VMEM((1, DIM), jnp.float32),
        pltpu.SemaphoreType.DMA,
        pltpu.SemaphoreType.DMA,
    ],
)
def _pooling_kernel(inputs_hbm, table_hbm, out_hbm, idx_v, idxg_v, rows_v,
                    out_v, t0_v, sem0, sem1):
    _pooling_body(inputs_hbm, table_hbm, out_hbm, idx_v, idxg_v, rows_v,
                  out_v, t0_v, sem0, sem1)


def kernel(inputs, table):
    tab_slab = _transpose_tc(table.T)
    view = tab_slab.reshape(TGRID * TSLABS * 4, DIM)
    return _pooling_kernel(inputs, view)
